# baseline (speedup 1.0000x reference)
import functools

import jax
import jax.numpy as jnp
from jax.experimental import pallas as pl
from jax.experimental.pallas import tpu as pltpu

_BS = 2


def _vq_body(z_ref, cb_ref, out_ref, idx_ref, loss_ref):
    zb = z_ref[...]
    cb = cb_ref[...]
    bs, d, nn = zb.shape
    k = cb.shape[0]
    xb = zb.transpose(0, 2, 1).reshape(bs * nn, d)
    m = bs * nn

    mm = jax.lax.dot_general(
        xb, cb, (((1,), (1,)), ((), ())),
        preferred_element_type=jnp.float32)
    x2 = jnp.sum(xb * xb, axis=1, keepdims=True)
    c2 = jnp.sum(cb * cb, axis=1)
    dist = (x2 - 2.0 * mm) + c2[None, :]

    minval = jnp.min(dist, axis=1, keepdims=True)
    iota = jax.lax.broadcasted_iota(jnp.int32, (m, k), 1)
    idx = jnp.min(jnp.where(dist == minval, iota, k), axis=1)
    idx_ref[...] = idx.reshape(bs, 1, nn)

    onehot = (iota == idx[:, None]).astype(jnp.bfloat16)
    outb = jax.lax.dot_general(
        cb.astype(jnp.bfloat16), onehot, (((0,), (1,)), ((), ())),
        preferred_element_type=jnp.float32)
    out_ref[...] = outb.reshape(d, bs, nn).transpose(1, 0, 2)

    loss_ref[...] = jnp.sum(minval).reshape(1, 1, 1)


@functools.partial(jax.jit, static_argnames=("interpret",))
def kernel(z, codebook, interpret=False):
    B, d, N = z.shape
    K = codebook.shape[0]
    G = B // _BS

    out, idx3, loss_sum = pl.pallas_call(
        _vq_body,
        grid=(G,),
        in_specs=[
            pl.BlockSpec((_BS, d, N), lambda b: (b, 0, 0)),
            pl.BlockSpec((K, d), lambda b: (0, 0)),
        ],
        out_specs=[
            pl.BlockSpec((_BS, d, N), lambda b: (b, 0, 0)),
            pl.BlockSpec((_BS, 1, N), lambda b: (b, 0, 0)),
            pl.BlockSpec((1, 1, 1), lambda b: (b, 0, 0)),
        ],
        out_shape=[
            jax.ShapeDtypeStruct((B, d, N), jnp.float32),
            jax.ShapeDtypeStruct((B, 1, N), jnp.int32),
            jax.ShapeDtypeStruct((G, 1, 1), jnp.float32),
        ],
        compiler_params=pltpu.CompilerParams(
            dimension_semantics=("parallel",)),
        interpret=interpret,
    )(z, codebook)

    commit_loss = 0.25 * jnp.sum(loss_sum) / (B * N * d)
    return out, idx3.reshape(B, N), commit_loss
